# Initial kernel scaffold; baseline (speedup 1.0000x reference)
import jax
import jax.numpy as jnp
from jax.experimental import pallas as pl


def kernel(hidden_states, te_w1, te_b1, te_w2, te_b2, tr_w1, tr_b1, tr_w2, tr_b2, pos_emb, se_w1, se_b1, se_w2, se_b2, attn_in_w, attn_in_b, attn_out_w, attn_out_b):
    raise NotImplementedError("write your pallas kernel here")


# speedup vs baseline: 1.4833x; 1.4833x over previous
import jax
import jax.numpy as jnp
import numpy as np
from jax.experimental import pallas as pl
from jax.experimental.pallas import tpu as pltpu

B = 1
S = 2048
H = 1024
E = 8
TOPK = 2
H4 = H // 4
NH = 8
DH = H // NH
CAP = int(B * S * 1.5 * TOPK / E)

HP = 1152
_f32 = jnp.float32
_SCALE = np.float32(1.0 / np.sqrt(np.float64(DH)))


def _dot(a, b):
    return jax.lax.dot_general(a, b, (((1,), (0,)), ((), ())),
                               preferred_element_type=_f32)


def _dot_rhst(a, bt):
    return jax.lax.dot_general(a, bt, (((1,), (1,)), ((), ())),
                               preferred_element_type=_f32)


def _k1_body(comb_ref, w1_ref, b1_ref, w2_ref, b2_ref, pe_ref,
             sw1_ref, sb1_ref, sw2_ref, sb2_ref, wqkv_ref, bqkv_ref,
             qkv_ref):
    x = comb_ref[...]
    h1 = jnp.maximum(_dot(x, w1_ref[...]) + b1_ref[...], 0.0)
    enc = _dot(h1, w2_ref[...]) + b2_ref[...]
    sea = x[:, (H + 1):(H + 2)]
    s1 = jnp.maximum(sea * sw1_ref[...] + sb1_ref[...], 0.0)
    sh = _dot(s1, sw2_ref[...]) + sb2_ref[...]
    sfull = jnp.concatenate([sh, sh, sh, sh], axis=1)
    enc = enc + pe_ref[...] + sfull
    qkv_ref[...] = _dot_rhst(enc, wqkv_ref[...]) + bqkv_ref[...]


def _k2_body(q_ref, k_ref, v_ref, o_ref, m_ref, l_ref):
    kb = pl.program_id(2)

    @pl.when(kb == 0)
    def _():
        o_ref[...] = jnp.zeros_like(o_ref)
        m_ref[...] = jnp.full_like(m_ref, -jnp.inf)
        l_ref[...] = jnp.zeros_like(l_ref)

    s = _dot_rhst(q_ref[...], k_ref[...]) * _SCALE
    bm = jnp.max(s, axis=1, keepdims=True)
    old_m = m_ref[...]
    new_m = jnp.maximum(old_m, bm)
    corr_e = jnp.where(old_m == new_m, 0.0, old_m - new_m)
    p = jnp.exp(s - new_m)
    bs = jnp.sum(p, axis=1, keepdims=True)
    old_l = l_ref[...]
    new_l = jnp.exp(corr_e) * old_l + bs
    acc = (jnp.exp(corr_e) * old_l) * o_ref[...]
    mm = _dot(p, v_ref[...])
    o_ref[...] = (acc + mm) * (1.0 / new_l)
    m_ref[...] = new_m
    l_ref[...] = new_l


def _k3a_body(o_ref, wo_ref, bo_ref, w1_ref, b1_ref, w2_ref, b2_ref,
              probs_ref, idx_ref, gates_ref):
    x = _dot_rhst(o_ref[...], wo_ref[...]) + bo_ref[...]
    h1 = jnp.maximum(_dot(x, w1_ref[...]) + b1_ref[...], 0.0)
    logits = (_dot(h1, w2_ref[...]) + b2_ref[...])[:, :E]
    m = jnp.max(logits, axis=1, keepdims=True)
    ex = jnp.exp(logits - m)
    probs = ex / jnp.sum(ex, axis=1, keepdims=True)
    probs_ref[...] = probs
    io = jax.lax.broadcasted_iota(jnp.int32, probs.shape, 1)
    v0 = jnp.max(probs, axis=1, keepdims=True)
    i0 = jnp.min(jnp.where(probs == v0, io, E), axis=1, keepdims=True)
    pm = jnp.where(io == i0, -1.0, probs)
    v1 = jnp.max(pm, axis=1, keepdims=True)
    i1 = jnp.min(jnp.where(pm == v1, io, E), axis=1, keepdims=True)
    idx_ref[...] = jnp.concatenate([i0, i1], axis=1)
    tv = jnp.concatenate([v0, v1], axis=1)
    gates_ref[...] = tv / jnp.sum(tv, axis=1, keepdims=True)


def _k3c_body(probs_ref, aux_ref):
    mp = jnp.sum(probs_ref[...], axis=0, keepdims=True) / _f32(S)
    aux_ref[...] = jnp.sum(mp * jnp.log(mp * _f32(E) + 1e-9),
                           axis=1, keepdims=True)


def _k3b_body(idx_ref, flat_ref, carry_ref):
    i = pl.program_id(0)

    @pl.when(i == 0)
    def _():
        carry_ref[...] = jnp.zeros_like(carry_ref)

    idx = idx_ref[...]
    tb = idx.shape[0]
    ioe = jax.lax.broadcasted_iota(jnp.int32, (tb, E), 1)
    oh0 = (ioe == idx[:, 0:1]).astype(_f32)
    oh1 = (ioe == idx[:, 1:2]).astype(_f32)
    cnt = oh0 + oh1
    r = jax.lax.broadcasted_iota(jnp.int32, (tb, tb), 0)
    c = jax.lax.broadcasted_iota(jnp.int32, (tb, tb), 1)
    tri = (c < r).astype(_f32)
    cum = _dot(tri, cnt) + carry_ref[...]
    carry_ref[...] = carry_ref[...] + jnp.sum(cnt, axis=0, keepdims=True)
    pos0 = jnp.sum(cum * oh0, axis=1, keepdims=True)
    pos1 = jnp.sum((cum + oh0) * oh1, axis=1, keepdims=True)
    f0 = jnp.where(pos0 < CAP, idx[:, 0:1] * CAP + pos0.astype(jnp.int32), -1)
    f1 = jnp.where(pos1 < CAP, idx[:, 1:2] * CAP + pos1.astype(jnp.int32), -1)
    flat_ref[...] = jnp.concatenate([f0, f1], axis=1)


def _k4_body(flat_ref, gates_ref, disp_ref, comb_ref):
    f = flat_ref[...]
    g = gates_ref[...]
    ts2 = f.shape[0]
    io = jax.lax.broadcasted_iota(jnp.int32, (ts2, E * CAP), 1)
    m0 = io == f[:, 0:1]
    m1 = io == f[:, 1:2]
    disp_ref[...] = m0.astype(_f32) + m1.astype(_f32)
    comb_ref[...] = (jnp.where(m0, g[:, 0:1], 0.0)
                     + jnp.where(m1, g[:, 1:2], 0.0))


def kernel(hidden_states, te_w1, te_b1, te_w2, te_b2, tr_w1, tr_b1, tr_w2,
           tr_b2, pos_emb, se_w1, se_b1, se_w2, se_b2, attn_in_w, attn_in_b,
           attn_out_w, attn_out_b):
    f32 = _f32
    hs = hidden_states[0]
    ts = jnp.arange(S, dtype=f32)
    sea = jnp.sin(ts * 2.0 * jnp.pi / 24.0)
    comb = jnp.concatenate(
        [hs, ts[:, None], sea[:, None],
         jnp.zeros((S, HP - H - 2), f32)], axis=1)
    w1p = jnp.concatenate(
        [te_w1, jnp.zeros((HP - H - 2, H), f32)], axis=0)
    pe = pos_emb[:S]
    w2p = jnp.concatenate([tr_w2, jnp.zeros((H, H - E), f32)], axis=1)
    b2p = jnp.concatenate([tr_b2, jnp.zeros((H - E,), f32)])

    TS = 256
    qkv = pl.pallas_call(
        _k1_body,
        grid=(S // TS,),
        in_specs=[
            pl.BlockSpec((TS, HP), lambda i: (i, 0)),
            pl.BlockSpec((HP, H), lambda i: (0, 0)),
            pl.BlockSpec((1, H), lambda i: (0, 0)),
            pl.BlockSpec((H, H), lambda i: (0, 0)),
            pl.BlockSpec((1, H), lambda i: (0, 0)),
            pl.BlockSpec((TS, H), lambda i: (i, 0)),
            pl.BlockSpec((1, H4), lambda i: (0, 0)),
            pl.BlockSpec((1, H4), lambda i: (0, 0)),
            pl.BlockSpec((H4, H4), lambda i: (0, 0)),
            pl.BlockSpec((1, H4), lambda i: (0, 0)),
            pl.BlockSpec((3 * H, H), lambda i: (0, 0)),
            pl.BlockSpec((1, 3 * H), lambda i: (0, 0)),
        ],
        out_specs=pl.BlockSpec((TS, 3 * H), lambda i: (i, 0)),
        out_shape=jax.ShapeDtypeStruct((S, 3 * H), f32),
    )(comb, w1p, te_b1[None], te_w2, te_b2[None], pe,
      se_w1, se_b1[None], se_w2, se_b2[None],
      attn_in_w, attn_in_b[None])

    QB = 1024
    KB = 1024
    o = pl.pallas_call(
        _k2_body,
        grid=(NH, S // QB, S // KB),
        in_specs=[
            pl.BlockSpec((QB, DH), lambda h, i, j: (i, h)),
            pl.BlockSpec((KB, DH), lambda h, i, j: (j, NH + h)),
            pl.BlockSpec((KB, DH), lambda h, i, j: (j, 2 * NH + h)),
        ],
        out_specs=pl.BlockSpec((QB, DH), lambda h, i, j: (i, h)),
        out_shape=jax.ShapeDtypeStruct((S, H), f32),
        scratch_shapes=[pltpu.VMEM((QB, 1), f32), pltpu.VMEM((QB, 1), f32)],
    )(qkv, qkv, qkv)

    RB = 512
    probs2, top_idx, gates = pl.pallas_call(
        _k3a_body,
        grid=(S // RB,),
        in_specs=[
            pl.BlockSpec((RB, H), lambda i: (i, 0)),
            pl.BlockSpec((H, H), lambda i: (0, 0)),
            pl.BlockSpec((1, H), lambda i: (0, 0)),
            pl.BlockSpec((H, H), lambda i: (0, 0)),
            pl.BlockSpec((1, H), lambda i: (0, 0)),
            pl.BlockSpec((H, H), lambda i: (0, 0)),
            pl.BlockSpec((1, H), lambda i: (0, 0)),
        ],
        out_specs=[
            pl.BlockSpec((RB, E), lambda i: (i, 0)),
            pl.BlockSpec((RB, 2), lambda i: (i, 0)),
            pl.BlockSpec((RB, 2), lambda i: (i, 0)),
        ],
        out_shape=[
            jax.ShapeDtypeStruct((S, E), f32),
            jax.ShapeDtypeStruct((S, 2), jnp.int32),
            jax.ShapeDtypeStruct((S, 2), f32),
        ],
    )(o, attn_out_w, attn_out_b[None], tr_w1, tr_b1[None],
      w2p, b2p[None])

    aux2 = pl.pallas_call(
        _k3c_body,
        out_shape=jax.ShapeDtypeStruct((1, 1), f32),
    )(probs2)

    TB = 128
    flat = pl.pallas_call(
        _k3b_body,
        grid=(S // TB,),
        in_specs=[pl.BlockSpec((TB, 2), lambda i: (i, 0))],
        out_specs=pl.BlockSpec((TB, 2), lambda i: (i, 0)),
        out_shape=jax.ShapeDtypeStruct((S, 2), jnp.int32),
        scratch_shapes=[pltpu.VMEM((1, E), f32)],
    )(top_idx)

    TS2 = 128
    dispatch, combine = pl.pallas_call(
        _k4_body,
        grid=(S // TS2,),
        in_specs=[
            pl.BlockSpec((TS2, 2), lambda i: (i, 0)),
            pl.BlockSpec((TS2, 2), lambda i: (i, 0)),
        ],
        out_specs=[
            pl.BlockSpec((TS2, E * CAP), lambda i: (i, 0)),
            pl.BlockSpec((TS2, E * CAP), lambda i: (i, 0)),
        ],
        out_shape=[
            jax.ShapeDtypeStruct((S, E * CAP), f32),
            jax.ShapeDtypeStruct((S, E * CAP), f32),
        ],
    )(flat, gates)

    dispatch = dispatch.reshape(B, S, E, CAP)
    combine = combine.reshape(B, S, E, CAP)
    probs = probs2[None]
    aux = aux2.reshape(())
    return dispatch, combine, probs, aux


# speedup vs baseline: 2.0094x; 1.3547x over previous
import jax
import jax.numpy as jnp
import numpy as np
from jax.experimental import pallas as pl
from jax.experimental.pallas import tpu as pltpu

B = 1
S = 2048
H = 1024
E = 8
TOPK = 2
H4 = H // 4
NH = 8
DH = H // NH
CAP = int(B * S * 1.5 * TOPK / E)

HP = 1152
_f32 = jnp.float32
_SCALE = np.float32(1.0 / np.sqrt(np.float64(DH)))


def _dot(a, b):
    return jax.lax.dot_general(a, b, (((1,), (0,)), ((), ())),
                               preferred_element_type=_f32)


def _dot_rhst(a, bt):
    return jax.lax.dot_general(a, bt, (((1,), (1,)), ((), ())),
                               preferred_element_type=_f32)


def _k1_body(comb_ref, w1_ref, b1_ref, w2_ref, b2_ref, pe_ref,
             sw1_ref, sb1_ref, sw2_ref, sb2_ref, wqkv_ref, bqkv_ref,
             qkv_ref):
    x = comb_ref[...]
    h1 = jnp.maximum(_dot(x, w1_ref[...]) + b1_ref[...], 0.0)
    enc = _dot(h1, w2_ref[...]) + b2_ref[...]
    sea = x[:, (H + 1):(H + 2)]
    s1 = jnp.maximum(sea * sw1_ref[...] + sb1_ref[...], 0.0)
    sh = _dot(s1, sw2_ref[...]) + sb2_ref[...]
    sfull = jnp.concatenate([sh, sh, sh, sh], axis=1)
    enc = enc + pe_ref[...] + sfull
    qkv_ref[...] = _dot_rhst(enc, wqkv_ref[...]) + bqkv_ref[...]


def _k2_body(q_ref, k_ref, v_ref, o_ref, m_ref, l_ref):
    kb = pl.program_id(2)

    @pl.when(kb == 0)
    def _():
        o_ref[...] = jnp.zeros_like(o_ref)
        m_ref[...] = jnp.full_like(m_ref, -jnp.inf)
        l_ref[...] = jnp.zeros_like(l_ref)

    s = _dot_rhst(q_ref[...], k_ref[...]) * _SCALE
    bm = jnp.max(s, axis=1, keepdims=True)
    old_m = m_ref[...]
    new_m = jnp.maximum(old_m, bm)
    corr_e = jnp.where(old_m == new_m, 0.0, old_m - new_m)
    p = jnp.exp(s - new_m)
    bs = jnp.sum(p, axis=1, keepdims=True)
    old_l = l_ref[...]
    new_l = jnp.exp(corr_e) * old_l + bs
    acc = (jnp.exp(corr_e) * old_l) * o_ref[...]
    mm = _dot(p, v_ref[...])
    o_ref[...] = (acc + mm) * (1.0 / new_l)
    m_ref[...] = new_m
    l_ref[...] = new_l


def _k3a_body(o_ref, wo_ref, bo_ref, w1_ref, b1_ref, w2_ref, b2_ref,
              probs_ref, idx_ref, gates_ref):
    x = _dot_rhst(o_ref[...], wo_ref[...]) + bo_ref[...]
    h1 = jnp.maximum(_dot(x, w1_ref[...]) + b1_ref[...], 0.0)
    logits = (_dot(h1, w2_ref[...]) + b2_ref[...])[:, :E]
    m = jnp.max(logits, axis=1, keepdims=True)
    ex = jnp.exp(logits - m)
    probs = ex / jnp.sum(ex, axis=1, keepdims=True)
    probs_ref[...] = probs
    io = jax.lax.broadcasted_iota(jnp.int32, probs.shape, 1)
    v0 = jnp.max(probs, axis=1, keepdims=True)
    i0 = jnp.min(jnp.where(probs == v0, io, E), axis=1, keepdims=True)
    pm = jnp.where(io == i0, -1.0, probs)
    v1 = jnp.max(pm, axis=1, keepdims=True)
    i1 = jnp.min(jnp.where(pm == v1, io, E), axis=1, keepdims=True)
    idx_ref[...] = jnp.concatenate([i0, i1], axis=1)
    tv = jnp.concatenate([v0, v1], axis=1)
    gates_ref[...] = tv / jnp.sum(tv, axis=1, keepdims=True)


def _k3c_body(probs_ref, aux_ref):
    mp = jnp.sum(probs_ref[...], axis=0, keepdims=True) / _f32(S)
    aux_ref[...] = jnp.sum(mp * jnp.log(mp * _f32(E) + 1e-9),
                           axis=1, keepdims=True)


def _k4_body(idx_ref, gates_ref, disp_ref, comb_ref, carry_ref):
    i = pl.program_id(0)

    @pl.when(i == 0)
    def _():
        carry_ref[...] = jnp.zeros_like(carry_ref)

    idx = idx_ref[...]
    g = gates_ref[...]
    tb = idx.shape[0]
    ioe = jax.lax.broadcasted_iota(jnp.int32, (tb, E), 1)
    oh0 = (ioe == idx[:, 0:1]).astype(_f32)
    oh1 = (ioe == idx[:, 1:2]).astype(_f32)
    cnt = oh0 + oh1
    r = jax.lax.broadcasted_iota(jnp.int32, (tb, tb), 0)
    c = jax.lax.broadcasted_iota(jnp.int32, (tb, tb), 1)
    tri = (c < r).astype(_f32)
    cum = _dot(tri, cnt) + carry_ref[...]
    carry_ref[...] = carry_ref[...] + jnp.sum(cnt, axis=0, keepdims=True)
    pos0 = jnp.sum(cum * oh0, axis=1, keepdims=True)
    pos1 = jnp.sum((cum + oh0) * oh1, axis=1, keepdims=True)
    p0 = jnp.where(pos0 < CAP, pos0.astype(jnp.int32), -1)
    p1 = jnp.where(pos1 < CAP, pos1.astype(jnp.int32), -1)

    ioe3 = jax.lax.broadcasted_iota(jnp.int32, (tb, E, CAP), 1)
    ioc3 = jax.lax.broadcasted_iota(jnp.int32, (tb, E, CAP), 2)
    m0 = (ioe3 == idx[:, 0:1, None]) & (ioc3 == p0[:, :, None])
    m1 = (ioe3 == idx[:, 1:2, None]) & (ioc3 == p1[:, :, None])
    disp_ref[...] = m0.astype(_f32) + m1.astype(_f32)
    comb_ref[...] = (jnp.where(m0, g[:, 0:1, None], 0.0)
                     + jnp.where(m1, g[:, 1:2, None], 0.0))


def kernel(hidden_states, te_w1, te_b1, te_w2, te_b2, tr_w1, tr_b1, tr_w2,
           tr_b2, pos_emb, se_w1, se_b1, se_w2, se_b2, attn_in_w, attn_in_b,
           attn_out_w, attn_out_b):
    f32 = _f32
    hs = hidden_states[0]
    ts = jnp.arange(S, dtype=f32)
    sea = jnp.sin(ts * 2.0 * jnp.pi / 24.0)
    comb = jnp.concatenate(
        [hs, ts[:, None], sea[:, None],
         jnp.zeros((S, HP - H - 2), f32)], axis=1)
    w1p = jnp.concatenate(
        [te_w1, jnp.zeros((HP - H - 2, H), f32)], axis=0)
    pe = pos_emb[:S]
    w2p = jnp.concatenate([tr_w2, jnp.zeros((H, H - E), f32)], axis=1)
    b2p = jnp.concatenate([tr_b2, jnp.zeros((H - E,), f32)])

    TS = 256
    qkv = pl.pallas_call(
        _k1_body,
        grid=(S // TS,),
        in_specs=[
            pl.BlockSpec((TS, HP), lambda i: (i, 0)),
            pl.BlockSpec((HP, H), lambda i: (0, 0)),
            pl.BlockSpec((1, H), lambda i: (0, 0)),
            pl.BlockSpec((H, H), lambda i: (0, 0)),
            pl.BlockSpec((1, H), lambda i: (0, 0)),
            pl.BlockSpec((TS, H), lambda i: (i, 0)),
            pl.BlockSpec((1, H4), lambda i: (0, 0)),
            pl.BlockSpec((1, H4), lambda i: (0, 0)),
            pl.BlockSpec((H4, H4), lambda i: (0, 0)),
            pl.BlockSpec((1, H4), lambda i: (0, 0)),
            pl.BlockSpec((3 * H, H), lambda i: (0, 0)),
            pl.BlockSpec((1, 3 * H), lambda i: (0, 0)),
        ],
        out_specs=pl.BlockSpec((TS, 3 * H), lambda i: (i, 0)),
        out_shape=jax.ShapeDtypeStruct((S, 3 * H), f32),
    )(comb, w1p, te_b1[None], te_w2, te_b2[None], pe,
      se_w1, se_b1[None], se_w2, se_b2[None],
      attn_in_w, attn_in_b[None])

    QB = 1024
    KB = 1024
    o = pl.pallas_call(
        _k2_body,
        grid=(NH, S // QB, S // KB),
        in_specs=[
            pl.BlockSpec((QB, DH), lambda h, i, j: (i, h)),
            pl.BlockSpec((KB, DH), lambda h, i, j: (j, NH + h)),
            pl.BlockSpec((KB, DH), lambda h, i, j: (j, 2 * NH + h)),
        ],
        out_specs=pl.BlockSpec((QB, DH), lambda h, i, j: (i, h)),
        out_shape=jax.ShapeDtypeStruct((S, H), f32),
        scratch_shapes=[pltpu.VMEM((QB, 1), f32), pltpu.VMEM((QB, 1), f32)],
    )(qkv, qkv, qkv)

    RB = 512
    probs2, top_idx, gates = pl.pallas_call(
        _k3a_body,
        grid=(S // RB,),
        in_specs=[
            pl.BlockSpec((RB, H), lambda i: (i, 0)),
            pl.BlockSpec((H, H), lambda i: (0, 0)),
            pl.BlockSpec((1, H), lambda i: (0, 0)),
            pl.BlockSpec((H, H), lambda i: (0, 0)),
            pl.BlockSpec((1, H), lambda i: (0, 0)),
            pl.BlockSpec((H, H), lambda i: (0, 0)),
            pl.BlockSpec((1, H), lambda i: (0, 0)),
        ],
        out_specs=[
            pl.BlockSpec((RB, E), lambda i: (i, 0)),
            pl.BlockSpec((RB, 2), lambda i: (i, 0)),
            pl.BlockSpec((RB, 2), lambda i: (i, 0)),
        ],
        out_shape=[
            jax.ShapeDtypeStruct((S, E), f32),
            jax.ShapeDtypeStruct((S, 2), jnp.int32),
            jax.ShapeDtypeStruct((S, 2), f32),
        ],
    )(o, attn_out_w, attn_out_b[None], tr_w1, tr_b1[None],
      w2p, b2p[None])

    aux2 = pl.pallas_call(
        _k3c_body,
        out_shape=jax.ShapeDtypeStruct((1, 1), f32),
    )(probs2)

    TB = 128
    dispatch, combine = pl.pallas_call(
        _k4_body,
        grid=(S // TB,),
        in_specs=[
            pl.BlockSpec((TB, 2), lambda i: (i, 0)),
            pl.BlockSpec((TB, 2), lambda i: (i, 0)),
        ],
        out_specs=[
            pl.BlockSpec((TB, E, CAP), lambda i: (i, 0, 0)),
            pl.BlockSpec((TB, E, CAP), lambda i: (i, 0, 0)),
        ],
        out_shape=[
            jax.ShapeDtypeStruct((S, E, CAP), f32),
            jax.ShapeDtypeStruct((S, E, CAP), f32),
        ],
        scratch_shapes=[pltpu.VMEM((1, E), f32)],
    )(top_idx, gates)

    dispatch = dispatch[None]
    combine = combine[None]
    probs = probs2[None]
    aux = aux2.reshape(())
    return dispatch, combine, probs, aux
